# baseline (speedup 1.0000x reference)
import functools

import jax
import jax.numpy as jnp
from jax import lax
from jax.experimental import pallas as pl
from jax.experimental.pallas import tpu as pltpu
from jax.experimental.pallas import tpu_sc as plsc

NIN = 26
VOCAB_P1 = 1001
D_MODEL = 64
BATCH = 1024
SEQ = 20

_info = plsc.get_sparse_core_info()
_NC, _NS = _info.num_cores, _info.num_subcores
_NW = _NC * _NS

_PAIRS = SEQ * NIN
_ITERS = (_PAIRS + _NW - 1) // _NW
_SUB = 128
_NSUB = BATCH // _SUB
_HALF = BATCH // 2


def _make_emb():
    mesh = plsc.VectorSubcoreMesh(core_axis_name="c", subcore_axis_name="s")

    @functools.partial(
        pl.kernel,
        mesh=mesh,
        out_type=jax.ShapeDtypeStruct((SEQ, NIN, D_MODEL, BATCH), jnp.float32),
        scratch_types=[
            pltpu.VMEM((BATCH,), jnp.int32),
            pltpu.VMEM((_SUB, D_MODEL), jnp.float32),
            pltpu.VMEM((_SUB, D_MODEL), jnp.float32),
            pltpu.VMEM((D_MODEL, _HALF), jnp.float32),
            pltpu.VMEM((D_MODEL, _HALF), jnp.float32),
            pltpu.SemaphoreType.DMA,
            pltpu.SemaphoreType.DMA,
            pltpu.SemaphoreType.DMA,
            pltpu.SemaphoreType.DMA,
        ],
        compiler_params=pltpu.CompilerParams(
            use_tc_tiling_on_sc=False, needs_layout_passes=False),
    )
    def emb(xt_hbm, tab_hbm, out_hbm, idx_v, gb0, gb1, tb0, tb1,
            gsem0, gsem1, osem0, osem1):
        wid = lax.axis_index("s") * _NC + lax.axis_index("c")
        lane = lax.iota(jnp.int32, 16)
        gbufs = (gb0, gb1)
        gsems = (gsem0, gsem1)
        tbufs = (tb0, tb1)
        osems = (osem0, osem1)

        def g_copy(c, buf, sem):
            return pltpu.make_async_copy(
                tab_hbm.at[idx_v.at[pl.ds(c * _SUB, _SUB)]], buf, sem)

        def pair_body(it, carry):
            p = it * _NW + wid

            @pl.when(p < _PAIRS)
            def _():
                l = p // NIN
                f = p - l * NIN
                xrow = f * SEQ + l
                pltpu.sync_copy(xt_hbm.at[xrow], idx_v)
                off = f * VOCAB_P1

                def add_off(j, carry2):
                    q = j * 16
                    idx_v[pl.ds(q, 16)] = idx_v[pl.ds(q, 16)] + off
                    return carry2

                lax.fori_loop(0, BATCH // 16, add_off, 0)

                g_copy(0, gb0, gsem0).start()
                for c in range(_NSUB):
                    h = c // (_NSUB // 2)
                    tb = tbufs[h]
                    g_copy(c, gbufs[c % 2], gsems[c % 2]).wait()
                    if c + 1 < _NSUB:
                        g_copy(c + 1, gbufs[(c + 1) % 2],
                               gsems[(c + 1) % 2]).start()
                    if c % (_NSUB // 2) == 0:
                        @pl.when(it > 0)
                        def _():
                            pltpu.make_async_copy(
                                tb, out_hbm.at[l, f, :, pl.ds(h * _HALF, _HALF)],
                                osems[h]).wait()
                    gb = gbufs[c % 2]
                    cb = (c % (_NSUB // 2)) * _SUB

                    def transpose_d(d, carry2):
                        col = jnp.full((16,), d, jnp.int32)
                        for g in range(_SUB // 16):
                            vec = plsc.load_gather(gb, [g * 16 + lane, col])
                            tb[d, pl.ds(cb + g * 16, 16)] = vec
                        return carry2

                    lax.fori_loop(0, D_MODEL, transpose_d, 0)
                    if c % (_NSUB // 2) == _NSUB // 2 - 1:
                        pltpu.make_async_copy(
                            tb, out_hbm.at[l, f, :, pl.ds(h * _HALF, _HALF)],
                            osems[h]).start()

            return carry

        lax.fori_loop(0, _ITERS, pair_body, 0)

        pltpu.make_async_copy(
            tb0, out_hbm.at[0, 0, :, pl.ds(0, _HALF)], osem0).wait()
        pltpu.make_async_copy(
            tb1, out_hbm.at[0, 0, :, pl.ds(_HALF, _HALF)], osem1).wait()

    return emb


def kernel(x, tables):
    xt = jnp.transpose(x.astype(jnp.int32), (2, 1, 0)).reshape(NIN * SEQ, BATCH)
    tab_flat = tables.reshape(NIN * VOCAB_P1, D_MODEL)
    out_t = _make_emb()(xt, tab_flat)
    return jnp.transpose(out_t, (3, 0, 1, 2))


# speedup vs baseline: 3.3827x; 3.3827x over previous
import functools

import jax
import jax.numpy as jnp
from jax import lax
from jax.experimental import pallas as pl
from jax.experimental.pallas import tpu as pltpu
from jax.experimental.pallas import tpu_sc as plsc

NIN = 26
VOCAB_P1 = 1001
VPAD = 1008
D_MODEL = 64
BATCH = 1024
SEQ = 20

_info = plsc.get_sparse_core_info()
_NC, _NS = _info.num_cores, _info.num_subcores
_NW = _NC * _NS
_DPW = D_MODEL // _NW


def _make_emb():
    mesh = plsc.VectorSubcoreMesh(core_axis_name="c", subcore_axis_name="s")

    @functools.partial(
        pl.kernel,
        mesh=mesh,
        out_type=jax.ShapeDtypeStruct((SEQ, NIN, D_MODEL, BATCH), jnp.float32),
        scratch_types=[
            pltpu.VMEM((SEQ, BATCH), jnp.int32),
            pltpu.VMEM((SEQ, BATCH), jnp.int32),
            pltpu.VMEM((VPAD,), jnp.float32),
            pltpu.VMEM((VPAD,), jnp.float32),
            pltpu.VMEM((SEQ, BATCH), jnp.float32),
            pltpu.VMEM((SEQ, BATCH), jnp.float32),
            pltpu.SemaphoreType.DMA,
            pltpu.SemaphoreType.DMA,
            pltpu.SemaphoreType.DMA,
            pltpu.SemaphoreType.DMA,
        ],
        compiler_params=pltpu.CompilerParams(
            use_tc_tiling_on_sc=False, needs_layout_passes=False),
    )
    def emb(xt_hbm, tabt_hbm, out_hbm, ib0, ib1, trb0, trb1, ob0, ob1,
            isem, tsem, osem0, osem1):
        wid = lax.axis_index("s") * _NC + lax.axis_index("c")
        ibufs = (ib0, ib1)
        trbufs = (trb0, trb1)
        obufs = (ob0, ob1)
        osems = (osem0, osem1)

        def i_copy(f, buf):
            return pltpu.make_async_copy(xt_hbm.at[f], buf, isem)

        def t_copy(f, d, buf):
            return pltpu.make_async_copy(
                tabt_hbm.at[f, d, pl.ds(0, VPAD)], buf, tsem)

        def o_copy(f, d, buf, sem):
            return pltpu.make_async_copy(buf, out_hbm.at[:, f, d], sem)

        i_copy(0, ib0).start()
        t_copy(0, wid, trb0).start()

        def one_field(f, ib, ib_next, first):
            i_copy(f, ib).wait()

            @pl.when(f + 1 < NIN)
            def _():
                i_copy(f + 1, ib_next).start()

            for j in range(_DPW):
                d = wid + j * _NW
                trb = trbufs[j]
                ob = obufs[j]
                t_copy(f, d, trb).wait()
                if j + 1 < _DPW:
                    t_copy(f, wid + (j + 1) * _NW, trbufs[j + 1]).start()
                else:
                    @pl.when(f + 1 < NIN)
                    def _():
                        t_copy(f + 1, wid, trbufs[0]).start()
                if not first:
                    o_copy(f, d, ob, osems[j]).wait()

                for l in range(SEQ):
                    @plsc.parallel_loop(0, BATCH // 16, unroll=8)
                    def gather_c(c):
                        ivec = ib[l, pl.ds(c * 16, 16)]
                        ob[l, pl.ds(c * 16, 16)] = plsc.load_gather(
                            trb, [ivec])

                o_copy(f, d, ob, osems[j]).start()

        one_field(0, ib0, ib1, True)

        def pair_body(p, carry):
            one_field(2 * p + 1, ib1, ib0, False)
            one_field(2 * p + 2, ib0, ib1, False)
            return carry

        lax.fori_loop(0, (NIN - 2) // 2, pair_body, 0)
        one_field(NIN - 1, ib1, ib0, False)
        for j in range(_DPW):
            o_copy(0, 0, obufs[j], osems[j]).wait()

    return emb


def kernel(x, tables):
    xt = jnp.transpose(x.astype(jnp.int32), (2, 1, 0))
    tabt = jnp.pad(jnp.transpose(tables, (0, 2, 1)),
                   ((0, 0), (0, 0), (0, VPAD - VOCAB_P1)))
    out_t = _make_emb()(xt, tabt)
    return jnp.transpose(out_t, (3, 0, 1, 2))


# speedup vs baseline: 5.8633x; 1.7333x over previous
import functools

import jax
import jax.numpy as jnp
from jax import lax
from jax.experimental import pallas as pl
from jax.experimental.pallas import tpu as pltpu
from jax.experimental.pallas import tpu_sc as plsc

NIN = 26
VOCAB_P1 = 1001
VPAD = 1008
D_MODEL = 64
BATCH = 1024
SEQ = 20

_info = plsc.get_sparse_core_info()
_NC, _NS = _info.num_cores, _info.num_subcores
_NW = _NC * _NS
_DPW = D_MODEL // _NW


def _make_emb():
    mesh = plsc.VectorSubcoreMesh(core_axis_name="c", subcore_axis_name="s")

    @functools.partial(
        pl.kernel,
        mesh=mesh,
        out_type=jax.ShapeDtypeStruct(
            (SEQ, NIN, D_MODEL // 8, BATCH // 128, 8, 128), jnp.float32),
        scratch_types=[
            pltpu.VMEM((SEQ, BATCH), jnp.int32),
            pltpu.VMEM((SEQ, BATCH), jnp.int32),
            pltpu.VMEM((VPAD,), jnp.float32),
            pltpu.VMEM((VPAD,), jnp.float32),
            pltpu.VMEM((SEQ, BATCH // 128, 128), jnp.float32),
            pltpu.VMEM((SEQ, BATCH // 128, 128), jnp.float32),
            pltpu.SemaphoreType.DMA,
            pltpu.SemaphoreType.DMA,
            pltpu.SemaphoreType.DMA,
            pltpu.SemaphoreType.DMA,
        ],
        compiler_params=pltpu.CompilerParams(
            use_tc_tiling_on_sc=False, needs_layout_passes=False),
    )
    def emb(xt_hbm, tabt_hbm, out_hbm, ib0, ib1, trb0, trb1, ob0, ob1,
            isem, tsem, osem0, osem1):
        wid = lax.axis_index("s") * _NC + lax.axis_index("c")
        ibufs = (ib0, ib1)
        trbufs = (trb0, trb1)
        obufs = (ob0, ob1)
        osems = (osem0, osem1)

        def i_copy(f, buf):
            return pltpu.make_async_copy(xt_hbm.at[f], buf, isem)

        def t_copy(f, d, buf):
            return pltpu.make_async_copy(
                tabt_hbm.at[f, d, pl.ds(0, VPAD)], buf, tsem)

        def o_copy(f, d, buf, sem):
            return pltpu.make_async_copy(
                buf, out_hbm.at[:, f, d // 8, :, d % 8], sem)

        i_copy(0, ib0).start()
        t_copy(0, wid, trb0).start()

        def one_field(f, ib, ib_next, first):
            i_copy(f, ib).wait()

            @pl.when(f + 1 < NIN)
            def _():
                i_copy(f + 1, ib_next).start()

            for j in range(_DPW):
                d = wid + j * _NW
                trb = trbufs[j]
                ob = obufs[j]
                t_copy(f, d, trb).wait()
                if j + 1 < _DPW:
                    t_copy(f, wid + (j + 1) * _NW, trbufs[j + 1]).start()
                else:
                    @pl.when(f + 1 < NIN)
                    def _():
                        t_copy(f + 1, wid, trbufs[0]).start()
                if not first:
                    o_copy(f, d, ob, osems[j]).wait()

                for l in range(SEQ):
                    @plsc.parallel_loop(0, BATCH // 16, unroll=8)
                    def gather_c(c):
                        ivec = ib[l, pl.ds(c * 16, 16)]
                        ob[l, c // 8, pl.ds((c % 8) * 16, 16)] = (
                            plsc.load_gather(trb, [ivec]))

                o_copy(f, d, ob, osems[j]).start()

        one_field(0, ib0, ib1, True)

        def pair_body(p, carry):
            one_field(2 * p + 1, ib1, ib0, False)
            one_field(2 * p + 2, ib0, ib1, False)
            return carry

        lax.fori_loop(0, (NIN - 2) // 2, pair_body, 0)
        one_field(NIN - 1, ib1, ib0, False)
        for j in range(_DPW):
            o_copy(0, 0, obufs[j], osems[j]).wait()

    return emb


def kernel(x, tables):
    xt = jnp.transpose(x.astype(jnp.int32), (2, 1, 0))
    tabt = jnp.pad(jnp.transpose(tables, (0, 2, 1)),
                   ((0, 0), (0, 0), (0, VPAD - VOCAB_P1)))
    out_t = _make_emb()(xt, tabt)
    out_p = jnp.transpose(out_t, (3, 5, 0, 1, 2, 4))
    return out_p.reshape(BATCH, SEQ, NIN, D_MODEL)
